# Initial kernel scaffold; baseline (speedup 1.0000x reference)
import jax
import jax.numpy as jnp
from jax.experimental import pallas as pl


def kernel(x, embed_weight):
    raise NotImplementedError("write your pallas kernel here")


# speedup vs baseline: 3.2709x; 3.2709x over previous
import jax
import jax.numpy as jnp
from jax.experimental import pallas as pl

B, T, C = 4, 8192, 1024
T_BLK = 256


def _add_kernel(x_ref, w_ref, o_ref):
    o_ref[...] = x_ref[...] + w_ref[...][None, :, :]


def kernel(x, embed_weight):
    w = embed_weight[:T]
    return pl.pallas_call(
        _add_kernel,
        grid=(T // T_BLK,),
        in_specs=[
            pl.BlockSpec((B, T_BLK, C), lambda t: (0, t, 0)),
            pl.BlockSpec((T_BLK, C), lambda t: (t, 0)),
        ],
        out_specs=pl.BlockSpec((B, T_BLK, C), lambda t: (0, t, 0)),
        out_shape=jax.ShapeDtypeStruct((B, T, C), x.dtype),
    )(x, w)


# speedup vs baseline: 3.2879x; 1.0052x over previous
import jax
import jax.numpy as jnp
from jax.experimental import pallas as pl

B, T, C = 4, 8192, 1024
T_BLK = 512


def _add_kernel(x_ref, w_ref, o_ref):
    o_ref[...] = x_ref[...] + w_ref[...][None, :, :]


def kernel(x, embed_weight):
    w = embed_weight[:T]
    return pl.pallas_call(
        _add_kernel,
        grid=(T // T_BLK,),
        in_specs=[
            pl.BlockSpec((B, T_BLK, C), lambda t: (0, t, 0)),
            pl.BlockSpec((T_BLK, C), lambda t: (t, 0)),
        ],
        out_specs=pl.BlockSpec((B, T_BLK, C), lambda t: (0, t, 0)),
        out_shape=jax.ShapeDtypeStruct((B, T, C), x.dtype),
    )(x, w)
